# baseline (speedup 1.0000x reference)
import functools
import jax
import jax.numpy as jnp
from jax import lax
from jax.experimental import pallas as pl
from jax.experimental.pallas import tpu as pltpu
from jax.experimental.pallas import tpu_sc as plsc

B, S, H, E, K = 4, 4096, 2048, 16, 2
AUX_COEF = 0.01
Z_COEF = 0.001
N = B * S
T = 1024
NBLK = N // T
NW = 32
TPT = N // NW
WPB = T // TPT
L = 16


def _tc_kernel(x_ref, w_ref, lt_ref, stats_ref):
    i = pl.program_id(0)

    lt = jax.lax.dot_general(
        w_ref[...], x_ref[...],
        dimension_numbers=(((1,), (1,)), ((), ())),
        preferred_element_type=jnp.float32)

    for j in range(WPB):
        lt_ref[j] = lt[:, j * TPT:(j + 1) * TPT]

    m = jnp.max(lt, axis=0, keepdims=True)
    denom = jnp.sum(jnp.exp(lt - m), axis=0, keepdims=True)
    z = m + jnp.log(denom)
    zsq = jnp.sum(z * z, axis=1, keepdims=True)

    @pl.when(i == 0)
    def _init():
        stats_ref[...] = jnp.zeros_like(stats_ref)

    stats_ref[0:1, 0:1] += zsq


def _sc_kernel(lt_hbm, rw_hbm, se_hbm, ps_hbm, cn_hbm,
               lt_v, rw_v, se_v, ps_v, cn_v):
    wid = lax.axis_index("s") * 2 + lax.axis_index("c")

    pltpu.sync_copy(lt_hbm.at[wid], lt_v)

    zero = jnp.zeros((L,), jnp.float32)
    for e in range(E):
        ps_v[e, :] = zero
        cn_v[e, :] = zero

    ecode = [jnp.full((L,), e, jnp.int32) for e in range(E)]
    bige = jnp.full((L,), E, jnp.int32)
    neg = jnp.full((L,), -1e30, jnp.float32)
    one = jnp.full((L,), 1.0, jnp.float32)

    def body(g, _):
        tok = pl.ds(g * L, L)
        v = [lt_v[e, tok] for e in range(E)]

        m = v[0]
        for e in range(1, E):
            m = jnp.maximum(m, v[e])

        ex = [jnp.exp(v[e] - m) for e in range(E)]
        denom = ex[0]
        for e in range(1, E):
            denom = denom + ex[e]
        recip = one / denom

        a1 = jnp.where(v[0] == m, ecode[0], bige)
        for e in range(1, E):
            a1 = jnp.minimum(a1, jnp.where(v[e] == m, ecode[e], bige))

        vm = [jnp.where(a1 == ecode[e], neg, v[e]) for e in range(E)]
        m2 = vm[0]
        for e in range(1, E):
            m2 = jnp.maximum(m2, vm[e])

        a2 = jnp.where(vm[0] == m2, ecode[0], bige)
        for e in range(1, E):
            a2 = jnp.minimum(a2, jnp.where(vm[e] == m2, ecode[e], bige))

        w1 = one / (one + jnp.exp(m2 - m))
        rw_v[0, tok] = w1
        rw_v[1, tok] = one - w1
        se_v[0, tok] = a1
        se_v[1, tok] = a2

        for e in range(E):
            ps_v[e, :] = ps_v[e, :] + ex[e] * recip
            hit1 = jnp.where(a1 == ecode[e], one, zero)
            hit2 = jnp.where(a2 == ecode[e], one, zero)
            cn_v[e, :] = cn_v[e, :] + hit1 + hit2
        return 0

    lax.fori_loop(0, TPT // L, body, 0)

    pltpu.sync_copy(rw_v, rw_hbm.at[wid])
    pltpu.sync_copy(se_v, se_hbm.at[wid])
    pltpu.sync_copy(ps_v, ps_hbm.at[wid])
    pltpu.sync_copy(cn_v, cn_hbm.at[wid])


def kernel(hidden_states, gate_w):
    x = hidden_states.reshape(N, H)
    lt, stats = pl.pallas_call(
        _tc_kernel,
        grid=(NBLK,),
        in_specs=[
            pl.BlockSpec((T, H), lambda i: (i, 0)),
            pl.BlockSpec((E, H), lambda i: (0, 0)),
        ],
        out_specs=[
            pl.BlockSpec((WPB, E, TPT), lambda i: (i, 0, 0)),
            pl.BlockSpec((8, 128), lambda i: (0, 0)),
        ],
        out_shape=[
            jax.ShapeDtypeStruct((NW, E, TPT), jnp.float32),
            jax.ShapeDtypeStruct((8, 128), jnp.float32),
        ],
    )(x, gate_w)

    mesh = plsc.VectorSubcoreMesh(core_axis_name="c", subcore_axis_name="s")
    sck = functools.partial(
        pl.kernel,
        out_type=[
            jax.ShapeDtypeStruct((NW, K, TPT), jnp.float32),
            jax.ShapeDtypeStruct((NW, K, TPT), jnp.int32),
            jax.ShapeDtypeStruct((NW, E, L), jnp.float32),
            jax.ShapeDtypeStruct((NW, E, L), jnp.float32),
        ],
        mesh=mesh,
        scratch_types=[
            pltpu.VMEM((E, TPT), jnp.float32),
            pltpu.VMEM((K, TPT), jnp.float32),
            pltpu.VMEM((K, TPT), jnp.int32),
            pltpu.VMEM((E, L), jnp.float32),
            pltpu.VMEM((E, L), jnp.float32),
        ],
    )(_sc_kernel)
    rw2, se2, ps, cn = sck(lt)

    routing_weights = jnp.stack(
        [rw2[:, 0, :].reshape(N), rw2[:, 1, :].reshape(N)], axis=-1
    ).reshape(B, S, K)
    selected_experts = jnp.stack(
        [se2[:, 0, :].reshape(N), se2[:, 1, :].reshape(N)], axis=-1
    ).reshape(B, S, K)

    ps_tot = jnp.sum(ps, axis=(0, 2))
    cn_tot = jnp.sum(cn, axis=(0, 2))
    aux = jnp.sum(ps_tot * cn_tot) * (float(E) / (float(N) * float(N)))
    loss = AUX_COEF * aux + Z_COEF * (stats[0, 0] / float(N))
    return routing_weights, selected_experts, loss


# speedup vs baseline: 1.5972x; 1.5972x over previous
import jax
import jax.numpy as jnp
from jax.experimental import pallas as pl

B, S, H, E, K = 4, 4096, 2048, 16, 2
AUX_COEF = 0.01
Z_COEF = 0.001
N = B * S
T = 1024
NBLK = N // T


def _router_kernel(x_ref, w_ref, rw_ref, se_ref, stats_ref):
    i = pl.program_id(0)

    lt = jax.lax.dot_general(
        w_ref[...], x_ref[...],
        dimension_numbers=(((1,), (1,)), ((), ())),
        preferred_element_type=jnp.float32)

    m = jnp.max(lt, axis=0, keepdims=True)
    ex = jnp.exp(lt - m)
    denom = jnp.sum(ex, axis=0, keepdims=True)
    z = m + jnp.log(denom)

    sidx = jax.lax.broadcasted_iota(jnp.int32, (E, T), 0)
    a1 = jnp.min(jnp.where(lt == m, sidx, E), axis=0, keepdims=True)
    mask1 = sidx == a1
    masked = jnp.where(mask1, -jnp.inf, lt)
    l2 = jnp.max(masked, axis=0, keepdims=True)
    a2 = jnp.min(jnp.where(masked == l2, sidx, E), axis=0, keepdims=True)
    mask2 = sidx == a2

    w1 = 1.0 / (1.0 + jnp.exp(l2 - m))
    rw_ref[...] = jnp.concatenate([w1, 1.0 - w1], axis=0)
    se_ref[...] = jnp.concatenate([a1, a2], axis=0)

    probs_sum = jnp.sum(ex * (1.0 / denom), axis=1, keepdims=True)
    counts = jnp.sum(mask1.astype(jnp.float32) + mask2.astype(jnp.float32),
                     axis=1, keepdims=True)
    zsq = jnp.sum(z * z, axis=1, keepdims=True)

    @pl.when(i == 0)
    def _init():
        stats_ref[...] = jnp.zeros_like(stats_ref)

    stats_ref[0:E, 0:1] += probs_sum
    stats_ref[0:E, 1:2] += counts
    stats_ref[0:1, 2:3] += zsq

    @pl.when(i == NBLK - 1)
    def _finish():
        ps = stats_ref[0:E, 0:1]
        cn = stats_ref[0:E, 1:2]
        zs = stats_ref[0:1, 2:3]
        aux = jnp.sum(cn * ps) * (float(E) / (float(N) * float(N)))
        loss = AUX_COEF * aux + Z_COEF * (zs / float(N))
        stats_ref[0:1, 3:4] = loss


def kernel(hidden_states, gate_w):
    x = hidden_states.reshape(N, H)
    rw, se, stats = pl.pallas_call(
        _router_kernel,
        grid=(NBLK,),
        in_specs=[
            pl.BlockSpec((T, H), lambda i: (i, 0)),
            pl.BlockSpec((E, H), lambda i: (0, 0)),
        ],
        out_specs=[
            pl.BlockSpec((K, T), lambda i: (0, i)),
            pl.BlockSpec((K, T), lambda i: (0, i)),
            pl.BlockSpec((E, 128), lambda i: (0, 0)),
        ],
        out_shape=[
            jax.ShapeDtypeStruct((K, N), jnp.float32),
            jax.ShapeDtypeStruct((K, N), jnp.int32),
            jax.ShapeDtypeStruct((E, 128), jnp.float32),
        ],
    )(x, gate_w)
    routing_weights = rw.T.reshape(B, S, K)
    selected_experts = se.T.reshape(B, S, K)
    return routing_weights, selected_experts, stats[0, 3]
